# baseline (speedup 1.0000x reference)
import functools

import jax
import jax.numpy as jnp
from jax import lax
from jax.experimental import pallas as pl
from jax.experimental.pallas import tpu as pltpu
from jax.experimental.pallas import tpu_sc as plsc

N_NODES = 10000
N_EDGES = 320000
D_FEAT = 128

NC = 2
NS = 16
NW = NC * NS

CHUNK = 80
EPT = 10240
NCHUNK = EPT // CHUNK
NPAIR = NCHUNK // 2
E_PAD = EPT * NW
VPF = D_FEAT // 16

RPT = 624
TAIL_OFF = NS * RPT
TAIL = N_NODES - TAIL_OFF


def _sc_spmm(cols_hbm, rows_hbm, vals_hbm, embeds_hbm, zeros_hbm, out_hbm,
             cols_v, rows_v, vals_v, gath_a, gath_b, acc_sh,
             sem_ga, sem_gb, sem_sa, sem_sb):
    cid = lax.axis_index("c")
    sid = lax.axis_index("s")
    wid = cid * NS + sid

    pltpu.sync_copy(cols_hbm.at[wid], cols_v)
    pltpu.sync_copy(rows_hbm.at[wid], rows_v)
    pltpu.sync_copy(vals_hbm.at[wid], vals_v)

    pltpu.sync_copy(zeros_hbm.at[pl.ds(sid * RPT, RPT)],
                    acc_sh.at[pl.ds(sid * RPT, RPT)])

    @pl.when(sid == NS - 1)
    def _():
        pltpu.sync_copy(zeros_hbm.at[pl.ds(TAIL_OFF, TAIL)],
                        acc_sh.at[pl.ds(TAIL_OFF, TAIL)])

    plsc.subcore_barrier()

    def fire_gather(c, buf, sem):
        return pltpu.async_copy(
            embeds_hbm.at[cols_v.at[pl.ds(c * CHUNK, CHUNK)]], buf, sem)

    def wait_gather(c, buf, sem):
        pltpu.make_async_copy(
            embeds_hbm.at[cols_v.at[pl.ds(c * CHUNK, CHUNK)]], buf, sem).wait()

    def fire_scatter(c, buf, sem):
        return pltpu.async_copy(
            buf, acc_sh.at[rows_v.at[c]], sem, add=True)

    def wait_scatter(c, buf, sem):
        pltpu.make_async_copy(
            buf, acc_sh.at[rows_v.at[c]], sem).wait()

    def scale(buf, c):
        base = c * CHUNK

        def g_body(g, _):
            vv16 = vals_v[pl.ds(base + g * 16, 16)]
            for i in range(16):
                e = g * 16 + i
                s = jnp.full((16,), vv16[i], jnp.float32)
                for k in range(VPF):
                    buf[e, pl.ds(k * 16, 16)] = buf[e, pl.ds(k * 16, 16)] * s
            return 0

        lax.fori_loop(0, CHUNK // 16, g_body, 0)

    def chunk_body(c, _):
        pltpu.async_copy(
            embeds_hbm.at[cols_v.at[pl.ds(c * CHUNK, CHUNK)]], gath_a,
            sem_ga).wait()
        scale(gath_a, c)
        pltpu.sync_copy(gath_a, acc_sh.at[rows_v.at[c]], add=True)
        return 0

    lax.fori_loop(0, NCHUNK, chunk_body, 0)

    plsc.subcore_barrier()

    pltpu.sync_copy(acc_sh.at[pl.ds(sid * RPT, RPT)],
                    out_hbm.at[cid, pl.ds(sid * RPT, RPT)])

    @pl.when(sid == NS - 1)
    def _():
        pltpu.sync_copy(acc_sh.at[pl.ds(TAIL_OFF, TAIL)],
                        out_hbm.at[cid, pl.ds(TAIL_OFF, TAIL)])


def _combine(a_ref, b_ref, o_ref):
    o_ref[...] = a_ref[...] + b_ref[...]


@jax.jit
def kernel(adj_indices, adj_values, embeds):
    pad = E_PAD - N_EDGES
    rows = jnp.concatenate(
        [adj_indices[0], jnp.zeros((pad,), jnp.int32)]).reshape(
            NW, NCHUNK, CHUNK)
    cols = jnp.concatenate(
        [adj_indices[1], jnp.zeros((pad,), jnp.int32)]).reshape(NW, EPT)
    vals = jnp.concatenate(
        [adj_values, jnp.zeros((pad,), jnp.float32)]).reshape(NW, EPT)
    hbm = functools.partial(pltpu.with_memory_space_constraint,
                            memory_space=pltpu.MemorySpace.HBM)
    rows, cols, vals = hbm(rows), hbm(cols), hbm(vals)
    zeros = hbm(jnp.zeros((N_NODES, D_FEAT), jnp.float32))

    mesh = plsc.VectorSubcoreMesh(core_axis_name="c", subcore_axis_name="s")
    partials = pl.kernel(
        _sc_spmm,
        out_type=jax.ShapeDtypeStruct((NC, N_NODES, D_FEAT), jnp.float32),
        mesh=mesh,
        scratch_types=[
            pltpu.VMEM((EPT,), jnp.int32),
            pltpu.VMEM((NCHUNK, CHUNK), jnp.int32),
            pltpu.VMEM((EPT,), jnp.float32),
            pltpu.VMEM((CHUNK, D_FEAT), jnp.float32),
            pltpu.VMEM((16, D_FEAT), jnp.float32),
            pltpu.VMEM_SHARED((N_NODES, D_FEAT), jnp.float32),
            pltpu.SemaphoreType.DMA,
            pltpu.SemaphoreType.DMA,
            pltpu.SemaphoreType.DMA,
            pltpu.SemaphoreType.DMA,
        ],
    )(cols, rows, vals, embeds, zeros)

    rows_blk = 1000
    out = pl.pallas_call(
        _combine,
        grid=(N_NODES // rows_blk,),
        in_specs=[pl.BlockSpec((rows_blk, D_FEAT), lambda i: (i, 0))] * 2,
        out_specs=pl.BlockSpec((rows_blk, D_FEAT), lambda i: (i, 0)),
        out_shape=jax.ShapeDtypeStruct((N_NODES, D_FEAT), jnp.float32),
    )(partials[0], partials[1])
    return out


# speedup vs baseline: 2.0539x; 2.0539x over previous
import functools

import jax
import jax.numpy as jnp
from jax import lax
from jax.experimental import pallas as pl
from jax.experimental.pallas import tpu as pltpu
from jax.experimental.pallas import tpu_sc as plsc

N_NODES = 10000
N_EDGES = 320000
D_FEAT = 128

NC = 2
NS = 16
NW = NC * NS

CHUNK = 80
EPT = 10000
NCHUNK = EPT // CHUNK
NPAIR = NCHUNK // 2
E_PAD = EPT * NW
VPF = D_FEAT // 16

RPT = 624
TAIL_OFF = NS * RPT
TAIL = N_NODES - TAIL_OFF


def _sc_spmm(cols_hbm, rows_hbm, vals_hbm, embeds_hbm, zeros_hbm, out_hbm,
             cols_v, rows_v, vals_v, gath_a, gath_b, acc_sh,
             sem_ga, sem_gb, sem_sa, sem_sb):
    cid = lax.axis_index("c")
    sid = lax.axis_index("s")
    wid = cid * NS + sid

    pltpu.sync_copy(cols_hbm.at[wid], cols_v)
    pltpu.sync_copy(rows_hbm.at[wid], rows_v)
    pltpu.sync_copy(vals_hbm.at[wid], vals_v)

    pltpu.sync_copy(zeros_hbm.at[pl.ds(sid * RPT, RPT)],
                    acc_sh.at[pl.ds(sid * RPT, RPT)])

    @pl.when(sid == NS - 1)
    def _():
        pltpu.sync_copy(zeros_hbm.at[pl.ds(TAIL_OFF, TAIL)],
                        acc_sh.at[pl.ds(TAIL_OFF, TAIL)])

    plsc.subcore_barrier()

    def fire_gather(c, buf, sem):
        return pltpu.async_copy(
            embeds_hbm.at[cols_v.at[pl.ds(c * CHUNK, CHUNK)]], buf, sem)

    def wait_gather(c, buf, sem):
        pltpu.make_async_copy(
            embeds_hbm.at[cols_v.at[pl.ds(c * CHUNK, CHUNK)]], buf, sem).wait()

    def fire_scatter(c, buf, sem):
        return pltpu.async_copy(
            buf, acc_sh.at[rows_v.at[c]], sem, add=True)

    def wait_scatter(c, buf, sem):
        pltpu.make_async_copy(
            buf, acc_sh.at[rows_v.at[c]], sem).wait()

    def scale(buf, c):
        base = c * CHUNK

        def g_body(g, _):
            vv16 = vals_v[pl.ds(base + g * 16, 16)]
            for i in range(16):
                e = g * 16 + i
                s = jnp.full((16,), vv16[i], jnp.float32)
                for k in range(VPF):
                    buf[e, pl.ds(k * 16, 16)] = buf[e, pl.ds(k * 16, 16)] * s
            return 0

        lax.fori_loop(0, CHUNK // 16, g_body, 0)

    def chunk_body(c, _):
        pltpu.async_copy(
            embeds_hbm.at[cols_v.at[pl.ds(c * CHUNK, CHUNK)]], gath_a,
            sem_ga).wait()
        scale(gath_a, c)
        pltpu.sync_copy(gath_a, acc_sh.at[rows_v.at[c]], add=True)
        return 0

    lax.fori_loop(0, NCHUNK, chunk_body, 0)

    plsc.subcore_barrier()

    pltpu.sync_copy(acc_sh.at[pl.ds(sid * RPT, RPT)],
                    out_hbm.at[cid, pl.ds(sid * RPT, RPT)])

    @pl.when(sid == NS - 1)
    def _():
        pltpu.sync_copy(acc_sh.at[pl.ds(TAIL_OFF, TAIL)],
                        out_hbm.at[cid, pl.ds(TAIL_OFF, TAIL)])


def _combine(a_ref, b_ref, o_ref):
    o_ref[...] = a_ref[...] + b_ref[...]


@jax.jit
def kernel(adj_indices, adj_values, embeds):
    rows = adj_indices[0].reshape(NW, NCHUNK, CHUNK)
    cols = adj_indices[1].reshape(NW, EPT)
    vals = adj_values.reshape(NW, EPT)
    hbm = functools.partial(pltpu.with_memory_space_constraint,
                            memory_space=pltpu.MemorySpace.HBM)
    rows, cols, vals = hbm(rows), hbm(cols), hbm(vals)
    zeros = hbm(jnp.zeros((N_NODES, D_FEAT), jnp.float32))

    mesh = plsc.VectorSubcoreMesh(core_axis_name="c", subcore_axis_name="s")
    partials = pl.kernel(
        _sc_spmm,
        out_type=jax.ShapeDtypeStruct((NC, N_NODES, D_FEAT), jnp.float32),
        mesh=mesh,
        scratch_types=[
            pltpu.VMEM((EPT,), jnp.int32),
            pltpu.VMEM((NCHUNK, CHUNK), jnp.int32),
            pltpu.VMEM((EPT,), jnp.float32),
            pltpu.VMEM((CHUNK, D_FEAT), jnp.float32),
            pltpu.VMEM((16, D_FEAT), jnp.float32),
            pltpu.VMEM_SHARED((N_NODES, D_FEAT), jnp.float32),
            pltpu.SemaphoreType.DMA,
            pltpu.SemaphoreType.DMA,
            pltpu.SemaphoreType.DMA,
            pltpu.SemaphoreType.DMA,
        ],
    )(cols, rows, vals, embeds, zeros)

    rows_blk = 1000
    out = pl.pallas_call(
        _combine,
        grid=(N_NODES // rows_blk,),
        in_specs=[pl.BlockSpec((rows_blk, D_FEAT), lambda i: (i, 0))] * 2,
        out_specs=pl.BlockSpec((rows_blk, D_FEAT), lambda i: (i, 0)),
        out_shape=jax.ShapeDtypeStruct((N_NODES, D_FEAT), jnp.float32),
    )(partials[0], partials[1])
    return out


# speedup vs baseline: 3.0119x; 1.4664x over previous
import functools

import jax
import jax.numpy as jnp
from jax import lax
from jax.experimental import pallas as pl
from jax.experimental.pallas import tpu as pltpu
from jax.experimental.pallas import tpu_sc as plsc

N_NODES = 10000
N_EDGES = 320000
D_FEAT = 128

NC = 2
NS = 16
NW = NC * NS

CHUNK = 80
EPT = 10000
NCHUNK = EPT // CHUNK
NPAIR = NCHUNK // 2
E_PAD = EPT * NW
VPF = D_FEAT // 16

RPT = 624
TAIL_OFF = NS * RPT
TAIL = N_NODES - TAIL_OFF


def _sc_spmm(cols_hbm, rows_hbm, vals_hbm, embeds_hbm, zeros_hbm, out_hbm,
             cols_v, rows_v, vals_v, gath_a, gath_b, acc_sh,
             sem_ga, sem_gb, sem_sa, sem_sb):
    cid = lax.axis_index("c")
    sid = lax.axis_index("s")
    wid = cid * NS + sid

    pltpu.sync_copy(cols_hbm.at[wid], cols_v)
    pltpu.sync_copy(rows_hbm.at[wid], rows_v)
    pltpu.sync_copy(vals_hbm.at[wid], vals_v)

    pltpu.sync_copy(zeros_hbm.at[pl.ds(sid * RPT, RPT)],
                    acc_sh.at[pl.ds(sid * RPT, RPT)])

    @pl.when(sid == NS - 1)
    def _():
        pltpu.sync_copy(zeros_hbm.at[pl.ds(TAIL_OFF, TAIL)],
                        acc_sh.at[pl.ds(TAIL_OFF, TAIL)])

    plsc.subcore_barrier()

    def fire_gather(c, buf, sem):
        return pltpu.async_copy(
            embeds_hbm.at[cols_v.at[pl.ds(c * CHUNK, CHUNK)]], buf, sem)

    def wait_gather(c, buf, sem):
        pltpu.make_async_copy(
            embeds_hbm.at[cols_v.at[pl.ds(c * CHUNK, CHUNK)]], buf, sem).wait()

    def fire_scatter(c, buf, sem):
        return pltpu.async_copy(
            buf, acc_sh.at[rows_v.at[pl.ds(c * CHUNK, CHUNK)]], sem, add=True)

    def wait_scatter(c, buf, sem):
        pltpu.make_async_copy(
            buf, acc_sh.at[rows_v.at[pl.ds(c * CHUNK, CHUNK)]], sem).wait()

    def scale(buf, c):
        base = c * CHUNK

        def g_body(g, _):
            vv16 = vals_v[pl.ds(base + g * 16, 16)]
            for i in range(16):
                e = g * 16 + i
                s = jnp.full((16,), vv16[i], jnp.float32)
                for k in range(VPF):
                    buf[e, pl.ds(k * 16, 16)] = buf[e, pl.ds(k * 16, 16)] * s
            return 0

        lax.fori_loop(0, CHUNK // 16, g_body, 0)

    fire_gather(0, gath_a, sem_ga)

    def pair_body(t, _):
        c0 = 2 * t
        c1 = 2 * t + 1
        wait_gather(c0, gath_a, sem_ga)

        @pl.when(t > 0)
        def _():
            wait_scatter(c1 - 2, gath_b, sem_sb)

        fire_gather(c1, gath_b, sem_gb)
        scale(gath_a, c0)
        fire_scatter(c0, gath_a, sem_sa)
        wait_gather(c1, gath_b, sem_gb)
        wait_scatter(c0, gath_a, sem_sa)
        fire_gather(c0 + 2, gath_a, sem_ga)
        scale(gath_b, c1)
        fire_scatter(c1, gath_b, sem_sb)
        return 0

    lax.fori_loop(0, NCHUNK // 2, pair_body, 0)

    wait_gather(NCHUNK - 1, gath_a, sem_ga)
    wait_scatter(NCHUNK - 2, gath_b, sem_sb)
    scale(gath_a, NCHUNK - 1)
    fire_scatter(NCHUNK - 1, gath_a, sem_sa)
    wait_scatter(NCHUNK - 1, gath_a, sem_sa)

    plsc.subcore_barrier()

    pltpu.sync_copy(acc_sh.at[pl.ds(sid * RPT, RPT)],
                    out_hbm.at[cid, pl.ds(sid * RPT, RPT)])

    @pl.when(sid == NS - 1)
    def _():
        pltpu.sync_copy(acc_sh.at[pl.ds(TAIL_OFF, TAIL)],
                        out_hbm.at[cid, pl.ds(TAIL_OFF, TAIL)])


def _combine(a_ref, b_ref, o_ref):
    o_ref[...] = a_ref[...] + b_ref[...]


@jax.jit
def kernel(adj_indices, adj_values, embeds):
    rows = adj_indices[0].reshape(NW, EPT)
    cols = adj_indices[1].reshape(NW, EPT)
    vals = adj_values.reshape(NW, EPT)
    hbm = functools.partial(pltpu.with_memory_space_constraint,
                            memory_space=pltpu.MemorySpace.HBM)
    rows, cols, vals = hbm(rows), hbm(cols), hbm(vals)
    zeros = hbm(jnp.zeros((N_NODES, D_FEAT), jnp.float32))

    mesh = plsc.VectorSubcoreMesh(core_axis_name="c", subcore_axis_name="s")
    partials = pl.kernel(
        _sc_spmm,
        out_type=jax.ShapeDtypeStruct((NC, N_NODES, D_FEAT), jnp.float32),
        mesh=mesh,
        scratch_types=[
            pltpu.VMEM((EPT,), jnp.int32),
            pltpu.VMEM((EPT,), jnp.int32),
            pltpu.VMEM((EPT,), jnp.float32),
            pltpu.VMEM((CHUNK, D_FEAT), jnp.float32),
            pltpu.VMEM((CHUNK, D_FEAT), jnp.float32),
            pltpu.VMEM_SHARED((N_NODES, D_FEAT), jnp.float32),
            pltpu.SemaphoreType.DMA,
            pltpu.SemaphoreType.DMA,
            pltpu.SemaphoreType.DMA,
            pltpu.SemaphoreType.DMA,
        ],
    )(cols, rows, vals, embeds, zeros)

    rows_blk = 1000
    out = pl.pallas_call(
        _combine,
        grid=(N_NODES // rows_blk,),
        in_specs=[pl.BlockSpec((rows_blk, D_FEAT), lambda i: (i, 0))] * 2,
        out_specs=pl.BlockSpec((rows_blk, D_FEAT), lambda i: (i, 0)),
        out_shape=jax.ShapeDtypeStruct((N_NODES, D_FEAT), jnp.float32),
    )(partials[0], partials[1])
    return out
